# Initial kernel scaffold; baseline (speedup 1.0000x reference)
import jax
import jax.numpy as jnp
from jax.experimental import pallas as pl


def kernel(X_cat, X_num, tables, W_num, b_num, W_final, b_final):
    raise NotImplementedError("write your pallas kernel here")


# speedup vs baseline: 1.6199x; 1.6199x over previous
import functools

import jax
import jax.numpy as jnp
from jax import lax
from jax.experimental import pallas as pl
from jax.experimental.pallas import tpu as pltpu
from jax.experimental.pallas import tpu_sc as plsc

N_CAT = 26
VOCAB = 100000
EMB = 64
N_NUM = 13
BATCH = 16384

NC = 2
NS = 16
NW = NC * NS
B_PER_W = BATCH // NW
SUB = 128
NSUB = B_PER_W // SUB


def _sc_gather_body(flat_idx, tables2d, cat_out, idx_v, rows_v, sem):
    wid = lax.axis_index("s") * NC + lax.axis_index("c")
    base = wid * B_PER_W

    @pl.loop(0, N_CAT)
    def _field(i):
        pltpu.sync_copy(flat_idx.at[i, wid], idx_v)
        cps = [
            pltpu.async_copy(tables2d.at[idx_v.at[j]], rows_v.at[j], sem)
            for j in range(NSUB)
        ]
        for c in cps:
            c.wait()
        for j in range(NSUB):
            pltpu.sync_copy(rows_v.at[j], cat_out.at[i, pl.ds(base + j * SUB, SUB)])


_sc_gather = pl.kernel(
    _sc_gather_body,
    out_type=jax.ShapeDtypeStruct((N_CAT, BATCH, EMB), jnp.float32),
    mesh=plsc.VectorSubcoreMesh(
        core_axis_name="c", subcore_axis_name="s", num_cores=NC, num_subcores=NS
    ),
    scratch_types=[
        pltpu.VMEM((NSUB, SUB), jnp.int32),
        pltpu.VMEM((NSUB, SUB, EMB), jnp.float32),
        pltpu.SemaphoreType.DMA,
    ],
    compiler_params=pltpu.CompilerParams(use_tc_tiling_on_sc=False),
)

BB = 1024


def _tc_proj_body(cat_ref, xn_ref, wn_ref, bn_ref, wfT_ref, bf_ref, out_ref):
    num = jnp.dot(xn_ref[...], wn_ref[...], preferred_element_type=jnp.float32)
    num = num + jnp.sum(bn_ref[...], axis=0, keepdims=True)
    acc = jnp.dot(num, wfT_ref[N_CAT * EMB :, :], preferred_element_type=jnp.float32)
    for i in range(N_CAT):
        acc = acc + jnp.dot(
            cat_ref[i], wfT_ref[i * EMB : (i + 1) * EMB, :],
            preferred_element_type=jnp.float32,
        )
    out_ref[...] = acc + bf_ref[...]


_tc_proj = pl.pallas_call(
    _tc_proj_body,
    grid=(BATCH // BB,),
    in_specs=[
        pl.BlockSpec((N_CAT, BB, EMB), lambda b: (0, b, 0)),
        pl.BlockSpec((BB, N_NUM), lambda b: (b, 0)),
        pl.BlockSpec((N_NUM, EMB), lambda b: (0, 0)),
        pl.BlockSpec((N_NUM, EMB), lambda b: (0, 0)),
        pl.BlockSpec((N_CAT * EMB + EMB, EMB), lambda b: (0, 0)),
        pl.BlockSpec((1, EMB), lambda b: (0, 0)),
    ],
    out_specs=pl.BlockSpec((BB, EMB), lambda b: (b, 0)),
    out_shape=jax.ShapeDtypeStruct((BATCH, EMB), jnp.float32),
)


def kernel(X_cat, X_num, tables, W_num, b_num, W_final, b_final):
    offs = (jnp.arange(N_CAT, dtype=jnp.int32) * VOCAB)[:, None]
    flat_idx = (X_cat.T + offs).reshape(N_CAT, NW, NSUB, SUB)
    tables2d = tables.reshape(N_CAT * VOCAB, EMB)
    cat = _sc_gather(flat_idx, tables2d)
    return _tc_proj(cat, X_num, W_num, b_num, W_final.T, b_final.reshape(1, EMB))
